# Initial kernel scaffold; baseline (speedup 1.0000x reference)
import jax
import jax.numpy as jnp
from jax.experimental import pallas as pl


def kernel(x, w, b):
    raise NotImplementedError("write your pallas kernel here")


# speedup vs baseline: 24.2175x; 24.2175x over previous
import functools

import jax
import jax.numpy as jnp
from jax import lax
from jax.experimental import pallas as pl
from jax.experimental.pallas import tpu as pltpu
from jax.experimental.pallas import tpu_sc as plsc

K = 9
GROUPS = 4


_BN = 448


def _knn_body(xr_ref, xc_ref, idx_ref):
    b = pl.program_id(0)
    n = xc_ref.shape[2]
    xc = xc_ref[0]
    sqc = jnp.sum(xc * xc, axis=0, keepdims=True)
    xcn = xc / jnp.maximum(jnp.sqrt(sqc), 1e-12)
    sqc_n = jnp.sum(xcn * xcn, axis=0)

    xr = xr_ref[0]
    sqr = jnp.sum(xr * xr, axis=1, keepdims=True)
    xrn = xr / jnp.maximum(jnp.sqrt(sqr), 1e-12)
    sqr_n = jnp.sum(xrn * xrn, axis=1, keepdims=True)

    inner = jnp.dot(xrn, xcn, preferred_element_type=jnp.float32)
    dist = sqr_n + (-2.0 * inner) + sqc_n[None, :]

    iota = lax.broadcasted_iota(jnp.int32, (_BN, n), 1)
    big = jnp.float32(3.0e38)
    cols = []
    for _ in range(K):
        m = jnp.min(dist, axis=1, keepdims=True)
        cand = jnp.where(dist == m, iota, n)
        idx = jnp.min(cand, axis=1)
        cols.append(idx)
        dist = jnp.where(cand == idx[:, None], big, dist)
    idx_ref[0] = jnp.stack(cols, axis=1) + b * n


def _knn_topk(xt, xc):
    b, n, c = xt.shape
    return pl.pallas_call(
        _knn_body,
        grid=(b, n // _BN),
        in_specs=[
            pl.BlockSpec((1, _BN, c), lambda i, j: (i, j, 0)),
            pl.BlockSpec((1, c, n), lambda i, j: (i, 0, 0)),
        ],
        out_specs=pl.BlockSpec((1, _BN, K), lambda i, j: (i, j, 0)),
        out_shape=jax.ShapeDtypeStruct((b, n, K), jnp.int32),
    )(xt, xc)


_NW = 32
_GR = 8


def _sc_maxdiff_body(rows_per_w, table_hbm, tpad_hbm, idx_hbm, out_hbm,
                     idx_v, ctr_v, out_v, rows0, sem0):
    wid = lax.axis_index("s") * 2 + lax.axis_index("c")
    base = wid * rows_per_w
    ng = rows_per_w // _GR
    pltpu.sync_copy(idx_hbm.at[pl.ds(base * K, rows_per_w * K)], idx_v)
    pltpu.sync_copy(table_hbm.at[pl.ds(base, rows_per_w)], ctr_v)

    def chunk_body(g, _):
        pltpu.async_copy(
            tpad_hbm.at[idx_v.at[pl.ds(g * (_GR * K), _GR * K)]],
            rows0, sem0).wait()

        def row_body(r, _):
            rr = g * _GR + r
            for cc in range(6):
                sl = pl.ds(cc * 16, 16)
                v = rows0[r * K, sl]
                for k in range(1, K):
                    v = jnp.maximum(v, rows0[r * K + k, sl])
                out_v[rr, sl] = v - ctr_v[rr, sl]
            return 0

        lax.fori_loop(0, _GR, row_body, 0)
        return 0

    lax.fori_loop(0, ng, chunk_body, 0)
    pltpu.sync_copy(out_v, out_hbm.at[pl.ds(base, rows_per_w)])


def _sc_maxdiff(table, table_pad, idx_flat):
    bn, c = table.shape
    cp = table_pad.shape[1]
    rows_per_w = bn // _NW
    mesh = plsc.VectorSubcoreMesh(core_axis_name="c", subcore_axis_name="s")
    kern = pl.kernel(
        functools.partial(_sc_maxdiff_body, rows_per_w),
        mesh=mesh,
        out_type=jax.ShapeDtypeStruct((bn, c), jnp.float32),
        scratch_types=[
            pltpu.VMEM((rows_per_w * K,), jnp.int32),
            pltpu.VMEM((rows_per_w, c), jnp.float32),
            pltpu.VMEM((rows_per_w, c), jnp.float32),
            pltpu.VMEM((_GR * K, cp), jnp.float32),
            pltpu.SemaphoreType.DMA,
        ],
    )
    return kern(table, table_pad, idx_flat)


def _conv_body(xt_ref, md_ref, wx_ref, wm_ref, b_ref, out_ref):
    acc = jnp.dot(xt_ref[0], wx_ref[...], preferred_element_type=jnp.float32)
    acc = acc + jnp.dot(md_ref[0], wm_ref[...], preferred_element_type=jnp.float32)
    out_ref[0] = jnp.maximum(acc + b_ref[...], 0.0)


def _conv_relu(xt, md, wx, wm, bias):
    b, n, c = xt.shape
    cout = wx.shape[1]
    return pl.pallas_call(
        _conv_body,
        grid=(b,),
        in_specs=[
            pl.BlockSpec((1, n, c), lambda i: (i, 0, 0)),
            pl.BlockSpec((1, n, c), lambda i: (i, 0, 0)),
            pl.BlockSpec((c, cout), lambda i: (0, 0)),
            pl.BlockSpec((c, cout), lambda i: (0, 0)),
            pl.BlockSpec((1, cout), lambda i: (0, 0)),
        ],
        out_specs=pl.BlockSpec((1, n, cout), lambda i: (i, 0, 0)),
        out_shape=jax.ShapeDtypeStruct((b, n, cout), jnp.float32),
    )(xt, md, wx, wm, bias)


def kernel(x, w, b):
    B, C, H, W = x.shape
    N = H * W
    cout = w.shape[0]

    xc = x.reshape(B, C, N)
    xt = jnp.transpose(xc, (0, 2, 1))

    gidx = _knn_topk(xt, xc)
    table = xt.reshape(B * N, C)
    table_pad = jnp.pad(table, ((0, 0), (0, (-C) % 128)))
    md = _sc_maxdiff(table, table_pad, gidx.reshape(-1))

    cpg = C // GROUPS
    opg = cout // GROUPS
    wg = w[:, :, 0, 0].reshape(GROUPS, opg, 2 * cpg)
    wx_g = wg[:, :, 0::2]
    wm_g = wg[:, :, 1::2]
    wx_full = jnp.zeros((C, cout), jnp.float32)
    wm_full = jnp.zeros((C, cout), jnp.float32)
    for g in range(GROUPS):
        wx_full = wx_full.at[g * cpg:(g + 1) * cpg, g * opg:(g + 1) * opg].set(
            jnp.transpose(wx_g[g]))
        wm_full = wm_full.at[g * cpg:(g + 1) * cpg, g * opg:(g + 1) * opg].set(
            jnp.transpose(wm_g[g]))

    out_t = _conv_relu(xt, md.reshape(B, N, C), wx_full, wm_full,
                       b.reshape(1, cout))
    return jnp.transpose(out_t, (0, 2, 1)).reshape(B, cout, H, W)
